# baseline (speedup 1.0000x reference)
import functools

import jax
import jax.numpy as jnp
from jax import lax
from jax.experimental import pallas as pl
from jax.experimental.pallas import tpu as pltpu
from jax.experimental.pallas import tpu_sc as plsc

ACTION_SIZE = 4
POINT_SIZE = 11
EMBED_DIM = 128
P_PAD = 16
BLK = 8192
LANES = 16


def _sims_body(x_ref, av_ref, out_ref):
    av = av_ref[:]
    n2sq = jnp.sum(av * av, axis=0, keepdims=True)
    avn = av * jax.lax.rsqrt(jnp.maximum(n2sq, 1e-30))
    out_ref[:] = jax.lax.dot_general(
        x_ref[:], avn, (((1,), (0,)), ((), ())),
        preferred_element_type=jnp.float32,
        precision=jax.lax.Precision.HIGHEST)


def _scores(embedded_words, av_pad):
    batch = embedded_words.shape[0]
    return pl.pallas_call(
        _sims_body,
        grid=(batch // BLK,),
        in_specs=[
            pl.BlockSpec((BLK, EMBED_DIM), lambda i: (i, 0)),
            pl.BlockSpec((EMBED_DIM, P_PAD), lambda i: (0, 0)),
        ],
        out_specs=pl.BlockSpec((BLK, P_PAD), lambda i: (i, 0)),
        out_shape=jax.ShapeDtypeStruct((batch, P_PAD), jnp.float32),
    )(embedded_words, av_pad)


def _argmax_onehot_sc(scores_flat, batch):
    info = plsc.get_sparse_core_info()
    n_workers = info.num_cores * info.num_subcores
    rpw = batch // n_workers
    mesh = plsc.VectorSubcoreMesh(core_axis_name="c", subcore_axis_name="s")

    @functools.partial(
        pl.kernel,
        out_type=jax.ShapeDtypeStruct((batch * ACTION_SIZE,), jnp.float32),
        mesh=mesh,
        scratch_types=[
            pltpu.VMEM((rpw * P_PAD,), jnp.float32),
            pltpu.VMEM((rpw * ACTION_SIZE,), jnp.float32),
        ],
        compiler_params=pltpu.CompilerParams(
            needs_layout_passes=False, skip_device_barrier=True),
    )
    def _sc(s_hbm, out_hbm, s_v, out_v):
        wid = lax.axis_index("s") * info.num_cores + lax.axis_index("c")
        base = wid * rpw
        pltpu.sync_copy(s_hbm.at[pl.ds(base * P_PAD, rpw * P_PAD)], s_v)

        def body(i, carry):
            row = i * LANES + lax.iota(jnp.int32, LANES)
            srow = row * P_PAD
            v = [plsc.load_gather(s_v, [srow + p])
                 for p in range(POINT_SIZE)]
            g0 = jnp.maximum(jnp.maximum(v[0], v[1]), jnp.maximum(v[2], v[3]))
            g1 = jnp.maximum(
                jnp.maximum(jnp.maximum(v[4], v[5]), jnp.maximum(v[6], v[7])),
                v[8])
            g2 = v[9]
            g3 = v[10]
            mx = jnp.maximum(jnp.maximum(g0, g1), jnp.maximum(g2, g3))
            o0 = g0 >= mx
            o1 = (g1 >= mx) & ~o0
            o2 = (g2 >= mx) & ~(o0 | o1)
            o3 = ~(o0 | o1 | o2)
            one = jnp.full((LANES,), 1.0, jnp.float32)
            zero = jnp.zeros((LANES,), jnp.float32)
            orow = row * ACTION_SIZE
            for a, o in enumerate((o0, o1, o2, o3)):
                plsc.store_scatter(out_v, [orow + a], jnp.where(o, one, zero))
            return carry

        lax.fori_loop(0, rpw // LANES, body, 0)
        pltpu.sync_copy(
            out_v, out_hbm.at[pl.ds(base * ACTION_SIZE, rpw * ACTION_SIZE)])

    return _sc(scores_flat)


def kernel(embedded_words, action_vectors):
    batch = embedded_words.shape[0]
    av = action_vectors[0]
    av_pad = jnp.pad(av, ((0, 0), (0, P_PAD - POINT_SIZE)))
    scores = _scores(embedded_words, av_pad)
    onehot = _argmax_onehot_sc(scores.reshape(-1), batch)
    return onehot.reshape(batch, ACTION_SIZE)


# speedup vs baseline: 1.0136x; 1.0136x over previous
import functools

import jax
import jax.numpy as jnp
from jax import lax
from jax.experimental import pallas as pl
from jax.experimental.pallas import tpu as pltpu
from jax.experimental.pallas import tpu_sc as plsc

ACTION_SIZE = 4
POINT_SIZE = 11
EMBED_DIM = 128
P_PAD = 16
BLK = 4096
LANES = 16


def _sims_body(x_ref, av_ref, out_ref):
    av = av_ref[:]
    n2sq = jnp.sum(av * av, axis=0, keepdims=True)
    avn = av * jax.lax.rsqrt(jnp.maximum(n2sq, 1e-30))
    out_ref[:] = jax.lax.dot_general(
        x_ref[:], avn, (((1,), (0,)), ((), ())),
        preferred_element_type=jnp.float32,
        precision=jax.lax.Precision.HIGHEST)


def _scores(embedded_words, av_pad):
    batch = embedded_words.shape[0]
    return pl.pallas_call(
        _sims_body,
        grid=(batch // BLK,),
        in_specs=[
            pl.BlockSpec((BLK, EMBED_DIM), lambda i: (i, 0)),
            pl.BlockSpec((EMBED_DIM, P_PAD), lambda i: (0, 0)),
        ],
        out_specs=pl.BlockSpec((BLK, P_PAD), lambda i: (i, 0)),
        out_shape=jax.ShapeDtypeStruct((batch, P_PAD), jnp.float32),
    )(embedded_words, av_pad)


def _argmax_onehot_sc(scores_flat, batch):
    info = plsc.get_sparse_core_info()
    num_cores = 1
    n_workers = num_cores * info.num_subcores
    rpw = batch // n_workers
    mesh = plsc.VectorSubcoreMesh(
        core_axis_name="c", subcore_axis_name="s", num_cores=num_cores)

    @functools.partial(
        pl.kernel,
        out_type=jax.ShapeDtypeStruct((batch * ACTION_SIZE,), jnp.float32),
        mesh=mesh,
        scratch_types=[
            pltpu.VMEM((rpw * P_PAD,), jnp.float32),
            pltpu.VMEM((rpw * ACTION_SIZE,), jnp.float32),
        ],
        compiler_params=pltpu.CompilerParams(
            needs_layout_passes=False, skip_device_barrier=True),
    )
    def _sc(s_hbm, out_hbm, s_v, out_v):
        wid = lax.axis_index("s") * num_cores + lax.axis_index("c")
        base = wid * rpw
        pltpu.sync_copy(s_hbm.at[pl.ds(base * P_PAD, rpw * P_PAD)], s_v)

        def body(i, carry):
            row = i * LANES + lax.iota(jnp.int32, LANES)
            srow = row * P_PAD
            v = [plsc.load_gather(s_v, [srow + p])
                 for p in range(POINT_SIZE)]
            g0 = jnp.maximum(jnp.maximum(v[0], v[1]), jnp.maximum(v[2], v[3]))
            g1 = jnp.maximum(
                jnp.maximum(jnp.maximum(v[4], v[5]), jnp.maximum(v[6], v[7])),
                v[8])
            g2 = v[9]
            g3 = v[10]
            mx = jnp.maximum(jnp.maximum(g0, g1), jnp.maximum(g2, g3))
            o0 = g0 >= mx
            o1 = (g1 >= mx) & ~o0
            o2 = (g2 >= mx) & ~(o0 | o1)
            o3 = ~(o0 | o1 | o2)
            one = jnp.full((LANES,), 1.0, jnp.float32)
            zero = jnp.zeros((LANES,), jnp.float32)
            orow = row * ACTION_SIZE
            for a, o in enumerate((o0, o1, o2, o3)):
                plsc.store_scatter(out_v, [orow + a], jnp.where(o, one, zero))
            return carry

        lax.fori_loop(0, rpw // LANES, body, 0)
        pltpu.sync_copy(
            out_v, out_hbm.at[pl.ds(base * ACTION_SIZE, rpw * ACTION_SIZE)])

    return _sc(scores_flat)


def kernel(embedded_words, action_vectors):
    batch = embedded_words.shape[0]
    av = action_vectors[0]
    av_pad = jnp.pad(av, ((0, 0), (0, P_PAD - POINT_SIZE)))
    scores = _scores(embedded_words, av_pad)
    onehot = _argmax_onehot_sc(scores.reshape(-1), batch)
    return onehot.reshape(batch, ACTION_SIZE)


# speedup vs baseline: 1.0158x; 1.0022x over previous
import functools

import jax
import jax.numpy as jnp
from jax import lax
from jax.experimental import pallas as pl
from jax.experimental.pallas import tpu as pltpu
from jax.experimental.pallas import tpu_sc as plsc

ACTION_SIZE = 4
POINT_SIZE = 11
EMBED_DIM = 128
P_PAD = 16
BLK = 4096
LANES = 16


def _sims_body(x_ref, av_ref, out_ref):
    av = av_ref[:]
    n2sq = jnp.sum(av * av, axis=0, keepdims=True)
    avn = av * jax.lax.rsqrt(jnp.maximum(n2sq, 1e-30))
    out_ref[:] = jax.lax.dot_general(
        x_ref[:], avn, (((1,), (0,)), ((), ())),
        preferred_element_type=jnp.float32,
        precision=jax.lax.Precision.HIGHEST)


def _scores(embedded_words, av_pad):
    batch = embedded_words.shape[0]
    return pl.pallas_call(
        _sims_body,
        grid=(batch // BLK,),
        in_specs=[
            pl.BlockSpec((BLK, EMBED_DIM), lambda i: (i, 0)),
            pl.BlockSpec((EMBED_DIM, P_PAD), lambda i: (0, 0)),
        ],
        out_specs=pl.BlockSpec((BLK, P_PAD), lambda i: (i, 0)),
        out_shape=jax.ShapeDtypeStruct((batch, P_PAD), jnp.float32),
    )(embedded_words, av_pad)


def _argmax_onehot_sc(scores_flat, batch):
    info = plsc.get_sparse_core_info()
    num_cores = info.num_cores
    n_workers = num_cores * info.num_subcores
    rpw = batch // n_workers
    mesh = plsc.VectorSubcoreMesh(
        core_axis_name="c", subcore_axis_name="s", num_cores=num_cores)

    @functools.partial(
        pl.kernel,
        out_type=jax.ShapeDtypeStruct((batch * ACTION_SIZE,), jnp.float32),
        mesh=mesh,
        scratch_types=[
            pltpu.VMEM((rpw * P_PAD,), jnp.float32),
            pltpu.VMEM((rpw * ACTION_SIZE,), jnp.float32),
        ],
        compiler_params=pltpu.CompilerParams(
            needs_layout_passes=False, skip_device_barrier=True),
    )
    def _sc(s_hbm, out_hbm, s_v, out_v):
        wid = lax.axis_index("s") * num_cores + lax.axis_index("c")
        base = wid * rpw
        pltpu.sync_copy(s_hbm.at[pl.ds(base * P_PAD, rpw * P_PAD)], s_v)

        def body(i, carry):
            row = i * LANES + lax.iota(jnp.int32, LANES)
            srow = row * P_PAD
            v = [plsc.load_gather(s_v, [srow + p])
                 for p in range(POINT_SIZE)]
            g0 = jnp.maximum(jnp.maximum(v[0], v[1]), jnp.maximum(v[2], v[3]))
            g1 = jnp.maximum(
                jnp.maximum(jnp.maximum(v[4], v[5]), jnp.maximum(v[6], v[7])),
                v[8])
            g2 = v[9]
            g3 = v[10]
            mx = jnp.maximum(jnp.maximum(g0, g1), jnp.maximum(g2, g3))
            o0 = g0 >= mx
            o1 = (g1 >= mx) & ~o0
            o2 = (g2 >= mx) & ~(o0 | o1)
            o3 = ~(o0 | o1 | o2)
            one = jnp.full((LANES,), 1.0, jnp.float32)
            zero = jnp.zeros((LANES,), jnp.float32)
            orow = row * ACTION_SIZE
            for a, o in enumerate((o0, o1, o2, o3)):
                plsc.store_scatter(out_v, [orow + a], jnp.where(o, one, zero))
            return carry

        lax.fori_loop(0, rpw // LANES, body, 0)
        pltpu.sync_copy(
            out_v, out_hbm.at[pl.ds(base * ACTION_SIZE, rpw * ACTION_SIZE)])

    return _sc(scores_flat)


def kernel(embedded_words, action_vectors):
    batch = embedded_words.shape[0]
    av = action_vectors[0]
    av_pad = jnp.pad(av, ((0, 0), (0, P_PAD - POINT_SIZE)))
    scores = _scores(embedded_words, av_pad)
    onehot = _argmax_onehot_sc(scores.reshape(-1), batch)
    return onehot.reshape(batch, ACTION_SIZE)


# speedup vs baseline: 1.0173x; 1.0015x over previous
import functools

import jax
import jax.numpy as jnp
from jax import lax
from jax.experimental import pallas as pl
from jax.experimental.pallas import tpu as pltpu
from jax.experimental.pallas import tpu_sc as plsc

ACTION_SIZE = 4
POINT_SIZE = 11
EMBED_DIM = 128
P_PAD = 16
BLK = 4096
LANES = 16


def _sims_body(x_ref, av_ref, out_ref):
    av = av_ref[:]
    n2sq = jnp.sum(av * av, axis=0, keepdims=True)
    avn = av * jax.lax.rsqrt(jnp.maximum(n2sq, 1e-30))
    out_ref[:] = jax.lax.dot_general(
        x_ref[:], avn, (((1,), (0,)), ((), ())),
        preferred_element_type=jnp.float32,
        precision=jax.lax.Precision.HIGHEST)


def _scores(embedded_words, av_pad):
    batch = embedded_words.shape[0]
    return pl.pallas_call(
        _sims_body,
        grid=(batch // BLK,),
        in_specs=[
            pl.BlockSpec((BLK, EMBED_DIM), lambda i: (i, 0)),
            pl.BlockSpec((EMBED_DIM, P_PAD), lambda i: (0, 0)),
        ],
        out_specs=pl.BlockSpec((BLK, P_PAD), lambda i: (i, 0)),
        out_shape=jax.ShapeDtypeStruct((batch, P_PAD), jnp.float32),
    )(embedded_words, av_pad)


def _argmax_onehot_sc(scores_flat, batch):
    info = plsc.get_sparse_core_info()
    num_cores = info.num_cores
    n_workers = num_cores * info.num_subcores
    rpw = batch // n_workers
    mesh = plsc.VectorSubcoreMesh(
        core_axis_name="c", subcore_axis_name="s", num_cores=num_cores)

    @functools.partial(
        pl.kernel,
        out_type=jax.ShapeDtypeStruct((batch * ACTION_SIZE,), jnp.float32),
        mesh=mesh,
        scratch_types=[
            pltpu.VMEM((rpw * P_PAD,), jnp.float32),
            pltpu.VMEM((rpw * ACTION_SIZE,), jnp.float32),
        ],
        compiler_params=pltpu.CompilerParams(needs_layout_passes=False),
    )
    def _sc(s_hbm, out_hbm, s_v, out_v):
        wid = lax.axis_index("s") * num_cores + lax.axis_index("c")
        base = wid * rpw
        pltpu.sync_copy(s_hbm.at[pl.ds(base * P_PAD, rpw * P_PAD)], s_v)

        def body(i, carry):
            row = i * LANES + lax.iota(jnp.int32, LANES)
            srow = row * P_PAD
            v = [plsc.load_gather(s_v, [srow + p])
                 for p in range(POINT_SIZE)]
            g0 = jnp.maximum(jnp.maximum(v[0], v[1]), jnp.maximum(v[2], v[3]))
            g1 = jnp.maximum(
                jnp.maximum(jnp.maximum(v[4], v[5]), jnp.maximum(v[6], v[7])),
                v[8])
            g2 = v[9]
            g3 = v[10]
            mx = jnp.maximum(jnp.maximum(g0, g1), jnp.maximum(g2, g3))
            o0 = g0 >= mx
            o1 = (g1 >= mx) & ~o0
            o2 = (g2 >= mx) & ~(o0 | o1)
            o3 = ~(o0 | o1 | o2)
            one = jnp.full((LANES,), 1.0, jnp.float32)
            zero = jnp.zeros((LANES,), jnp.float32)
            orow = row * ACTION_SIZE
            for a, o in enumerate((o0, o1, o2, o3)):
                plsc.store_scatter(out_v, [orow + a], jnp.where(o, one, zero))
            return carry

        lax.fori_loop(0, rpw // LANES, body, 0)
        pltpu.sync_copy(
            out_v, out_hbm.at[pl.ds(base * ACTION_SIZE, rpw * ACTION_SIZE)])

    return _sc(scores_flat)


def kernel(embedded_words, action_vectors):
    batch = embedded_words.shape[0]
    av = action_vectors[0]
    av_pad = jnp.pad(av, ((0, 0), (0, P_PAD - POINT_SIZE)))
    scores = _scores(embedded_words, av_pad)
    onehot = _argmax_onehot_sc(scores.reshape(-1), batch)
    return onehot.reshape(batch, ACTION_SIZE)
